# baseline (speedup 1.0000x reference)
import functools

import jax
import jax.numpy as jnp
from jax import lax
from jax.experimental import pallas as pl
from jax.experimental.pallas import tpu as pltpu
from jax.experimental.pallas import tpu_sc as plsc

_NUM_NODES = 100000
_EMBED_DIM = 32
_TABLE_WORDS = _NUM_NODES * _EMBED_DIM
_NUM_WORKERS = 32
_WORDS_PER_WORKER = _TABLE_WORDS // _NUM_WORKERS


def _broadcast_copy(batch):
    mesh = plsc.VectorSubcoreMesh(core_axis_name="c", subcore_axis_name="s")

    @functools.partial(
        pl.kernel,
        mesh=mesh,
        out_type=jax.ShapeDtypeStruct((batch * _TABLE_WORDS,), jnp.float32),
        scratch_types=[
            pltpu.VMEM((_WORDS_PER_WORKER,), jnp.float32),
            pltpu.SemaphoreType.DMA,
            pltpu.SemaphoreType.DMA,
        ],
    )
    def k(table_hbm, out_hbm, buf, sem_in, sem_out):
        wid = lax.axis_index("s") * 2 + lax.axis_index("c")
        base = wid * _WORDS_PER_WORKER
        pltpu.async_copy(
            table_hbm.at[pl.ds(base, _WORDS_PER_WORKER)], buf, sem_in
        ).wait()
        copies = [
            pltpu.async_copy(
                buf,
                out_hbm.at[pl.ds(b * _TABLE_WORDS + base, _WORDS_PER_WORKER)],
                sem_out,
            )
            for b in range(batch)
        ]
        for c in copies:
            c.wait()

    return k


def kernel(node_values, embeddings):
    batch = node_values.shape[0]
    flat = _broadcast_copy(batch)(embeddings.reshape(-1))
    return flat.reshape(batch, _NUM_NODES, _EMBED_DIM)


# speedup vs baseline: 1.0018x; 1.0018x over previous
import functools

import jax
import jax.numpy as jnp
from jax import lax
from jax.experimental import pallas as pl
from jax.experimental.pallas import tpu as pltpu
from jax.experimental.pallas import tpu_sc as plsc

_NUM_NODES = 100000
_EMBED_DIM = 32
_NUM_WORKERS = 32
_ROWS = _NUM_NODES // _NUM_WORKERS


def _broadcast_copy(batch):
    mesh = plsc.VectorSubcoreMesh(core_axis_name="c", subcore_axis_name="s")

    @functools.partial(
        pl.kernel,
        mesh=mesh,
        out_type=jax.ShapeDtypeStruct((batch, _NUM_NODES, _EMBED_DIM),
                                      jnp.float32),
        scratch_types=[
            pltpu.VMEM((_ROWS, _EMBED_DIM), jnp.float32),
            pltpu.SemaphoreType.DMA,
            pltpu.SemaphoreType.DMA,
        ],
        compiler_params=pltpu.CompilerParams(use_tc_tiling_on_sc=False),
    )
    def k(table_hbm, out_hbm, buf, sem_in, sem_out):
        wid = lax.axis_index("s") * 2 + lax.axis_index("c")
        base = wid * _ROWS
        pltpu.async_copy(
            table_hbm.at[pl.ds(base, _ROWS), :], buf, sem_in
        ).wait()
        copies = [
            pltpu.async_copy(
                buf, out_hbm.at[b, pl.ds(base, _ROWS), :], sem_out
            )
            for b in range(batch)
        ]
        for c in copies:
            c.wait()

    return k


def kernel(node_values, embeddings):
    batch = node_values.shape[0]
    return _broadcast_copy(batch)(embeddings)


# speedup vs baseline: 1.1883x; 1.1861x over previous
import functools

import jax
import jax.numpy as jnp
from jax import lax
from jax.experimental import pallas as pl
from jax.experimental.pallas import tpu as pltpu
from jax.experimental.pallas import tpu_sc as plsc

_NUM_NODES = 100000
_EMBED_DIM = 32
_NUM_WORKERS = 32
_CHUNK = 784
_CHUNKS_PER_WORKER = 4
_STRIDE = _CHUNK * _CHUNKS_PER_WORKER
_LAST_BASE = _NUM_NODES - _CHUNK


def _broadcast_copy(batch):
    mesh = plsc.VectorSubcoreMesh(core_axis_name="c", subcore_axis_name="s")

    @functools.partial(
        pl.kernel,
        mesh=mesh,
        out_type=jax.ShapeDtypeStruct((batch, _NUM_NODES, _EMBED_DIM),
                                      jnp.float32),
        scratch_types=[
            pltpu.VMEM((_CHUNK, _EMBED_DIM), jnp.float32),
            pltpu.SemaphoreType.DMA,
            pltpu.SemaphoreType.DMA,
        ],
    )
    def k(table_hbm, out_hbm, buf, sem_in, sem_out):
        wid = lax.axis_index("s") * 2 + lax.axis_index("c")
        for j in range(_CHUNKS_PER_WORKER):
            base = jnp.minimum(wid * _STRIDE + j * _CHUNK, _LAST_BASE)
            pltpu.async_copy(
                table_hbm.at[pl.ds(base, _CHUNK), :], buf, sem_in
            ).wait()
            copies = [
                pltpu.async_copy(
                    buf, out_hbm.at[b, pl.ds(base, _CHUNK), :], sem_out
                )
                for b in range(batch)
            ]
            for c in copies:
                c.wait()

    return k


def kernel(node_values, embeddings):
    batch = node_values.shape[0]
    return _broadcast_copy(batch)(embeddings)
